# baseline (speedup 1.0000x reference)
import jax
import jax.numpy as jnp
from jax import lax
from jax.experimental import pallas as pl
from jax.experimental.pallas import tpu as pltpu
from jax.experimental.pallas import tpu_sc as plsc

N = 2048
D = 64
C = 16
DP = 128
E2 = 32768
NW = 32
CPT = 8
CHUNK = 128
BN = 128
EB = 1024

_HIGHEST = lax.Precision.HIGHEST


def _sigmoid(x):
    return 1.0 / (1.0 + jnp.exp(-x))


def _softplus(x):
    return jnp.maximum(x, 0.0) + jnp.log(1.0 + jnp.exp(-jnp.abs(x)))


def _tc1_body(w_ref, b_ref, a_ref, out_ref):
    val = jnp.dot(w_ref[...], a_ref[...], preferred_element_type=jnp.float32,
                  precision=_HIGHEST) + b_ref[...]
    ones = jnp.ones((BN, 1), jnp.float32)
    pad = jnp.zeros((BN, DP - D - 1), jnp.float32)
    out_ref[...] = jnp.concatenate([val, ones, pad], axis=1)


def _tc1_call(s2p_w2, s2p_b2, alpha_diag):
    return pl.pallas_call(
        _tc1_body,
        grid=(N // BN,),
        in_specs=[
            pl.BlockSpec((BN, D * D), lambda g: (g, 0)),
            pl.BlockSpec((BN, D), lambda g: (g, 0)),
            pl.BlockSpec((D * D, D), lambda g: (0, 0)),
        ],
        out_specs=pl.BlockSpec((BN, DP), lambda g: (g, 0)),
        out_shape=jax.ShapeDtypeStruct((N, DP), jnp.float32),
    )(s2p_w2, s2p_b2, alpha_diag)


def _sc1_body(phi_hbm, src_hbm, dst_hbm, out_hbm, src_v, dst_v, buf, acc, sem):
    cid = lax.axis_index("c")
    sid = lax.axis_index("s")
    wid = cid * 16 + sid
    zero = jnp.zeros((16,), jnp.float32)

    def zrow(i, carry):
        for c in range(DP // 16):
            buf[i, pl.ds(c * 16, 16)] = zero
        return carry

    lax.fori_loop(0, CHUNK, zrow, 0)
    pltpu.sync_copy(buf, acc.at[pl.ds(sid * CHUNK, CHUNK)])
    pltpu.sync_copy(src_hbm.at[wid], src_v)
    pltpu.sync_copy(dst_hbm.at[wid], dst_v)
    plsc.subcore_barrier()
    for j in range(CPT):
        pltpu.async_copy(phi_hbm.at[src_v.at[j]], buf, sem).wait()
        pltpu.sync_copy(buf, acc.at[dst_v.at[j]], add=True)
    plsc.subcore_barrier()
    pltpu.sync_copy(acc.at[pl.ds(sid * CHUNK, CHUNK)], buf)
    pltpu.sync_copy(buf, out_hbm.at[cid, pl.ds(sid * CHUNK, CHUNK)])


def _sc1_call(phi_ext, src3, dst3):
    mesh = plsc.VectorSubcoreMesh(core_axis_name="c", subcore_axis_name="s")
    return pl.kernel(
        _sc1_body,
        out_type=jax.ShapeDtypeStruct((2, N, DP), jnp.float32),
        mesh=mesh,
        scratch_types=[
            pltpu.VMEM((CPT, CHUNK), jnp.int32),
            pltpu.VMEM((CPT, CHUNK), jnp.int32),
            pltpu.VMEM((CHUNK, DP), jnp.float32),
            pltpu.VMEM_SHARED((N, DP), jnp.float32),
            pltpu.SemaphoreType.DMA,
        ],
    )(phi_ext, src3, dst3)


def _tc2_body(phi_ext_ref, agg2_ref, alpha_ref, s2b_w_ref, s2b_b_ref,
              wihn_ref, bihn_ref, bhhn_ref, wihc_ref, bihc_ref, bhhc_ref,
              pm_ref, pmb_ref, ps_ref, psb_ref, bm_ref, bmb_ref, bs_ref,
              bsb_ref, epsp_ref, epsb_ref, dec_ref, phis_ref, bd_ref):
    phi_prior = phi_ext_ref[:, :D]
    acc = agg2_ref[0] + agg2_ref[1]
    agg = acc[:, :D]
    deg = acc[:, D:D + 1]
    ctx = jnp.where(deg == 0.0, phi_prior, agg) / jnp.maximum(deg, 1.0)

    x = jnp.concatenate([phi_prior, ctx], axis=1)
    gi = jnp.dot(x, wihn_ref[...], preferred_element_type=jnp.float32,
                 precision=_HIGHEST) + bihn_ref[...]
    bhh = bhhn_ref[...]
    r = _sigmoid(gi[:, :D] + bhh[:, :D])
    z = _sigmoid(gi[:, D:2 * D] + bhh[:, D:2 * D])
    n = jnp.tanh(gi[:, 2 * D:] + r * bhh[:, 2 * D:])
    h_phi = (1.0 - z) * n
    phi_mean = jnp.dot(h_phi, pm_ref[...], preferred_element_type=jnp.float32,
                       precision=_HIGHEST) + pmb_ref[...]
    phi_std = _softplus(jnp.dot(h_phi, ps_ref[...],
                                preferred_element_type=jnp.float32,
                                precision=_HIGHEST) + psb_ref[...])
    phis = phi_mean + phi_std * epsp_ref[...]
    phis_ref[...] = jnp.concatenate(
        [phis, jnp.zeros((N, DP - D), jnp.float32)], axis=1)

    bp = jnp.sum(s2b_w_ref[...] * alpha_ref[...][None], axis=2) + s2b_b_ref[...]
    xc = jnp.concatenate([bp, bp], axis=1)
    gic = jnp.dot(xc, wihc_ref[...], preferred_element_type=jnp.float32,
                  precision=_HIGHEST) + bihc_ref[...]
    bhc = bhhc_ref[...]
    rc = _sigmoid(gic[:, :D] + bhc[:, :D])
    zc = _sigmoid(gic[:, D:2 * D] + bhc[:, D:2 * D])
    nc = jnp.tanh(gic[:, 2 * D:] + rc * bhc[:, 2 * D:])
    h_beta = (1.0 - zc) * nc
    beta_mean = jnp.dot(h_beta, bm_ref[...], preferred_element_type=jnp.float32,
                        precision=_HIGHEST) + bmb_ref[...]
    beta_std = _softplus(jnp.dot(h_beta, bs_ref[...],
                                 preferred_element_type=jnp.float32,
                                 precision=_HIGHEST) + bsb_ref[...])
    beta_sample = beta_mean + beta_std * epsb_ref[...]
    bd_ref[...] = jnp.dot(beta_sample, dec_ref[...],
                          preferred_element_type=jnp.float32,
                          precision=_HIGHEST).astype(jnp.bfloat16)


def _tc2_call(phi_ext, agg2, alpha, s2b_w3, s2b_b2, wih_nT, bih_n2, bhh_n2,
              wih_cT, bih_c2, bhh_c2, pm_wT, pm_b2, ps_wT, ps_b2, bm_wT,
              bm_b2, bs_wT, bs_b2, eps_phi, eps_beta, dec_wT):
    return pl.pallas_call(
        _tc2_body,
        out_shape=(
            jax.ShapeDtypeStruct((N, DP), jnp.float32),
            jax.ShapeDtypeStruct((C, N), jnp.bfloat16),
        ),
    )(phi_ext, agg2, alpha, s2b_w3, s2b_b2, wih_nT, bih_n2, bhh_n2,
      wih_cT, bih_c2, bhh_c2, pm_wT, pm_b2, ps_wT, ps_b2, bm_wT, bm_b2,
      bs_wT, bs_b2, eps_phi, eps_beta, dec_wT)


def _sc2_body(phi_hbm, src_hbm, dst_hbm, out_hbm, src_v, dst_v, a_v, b_v,
              p_v, sem_a, sem_b):
    cid = lax.axis_index("c")
    sid = lax.axis_index("s")
    wid = cid * 16 + sid
    pltpu.sync_copy(src_hbm.at[wid], src_v)
    pltpu.sync_copy(dst_hbm.at[wid], dst_v)
    base = wid * (CPT * CHUNK)
    for j in range(CPT):
        cp_a = pltpu.async_copy(phi_hbm.at[src_v.at[j]], a_v, sem_a)
        cp_b = pltpu.async_copy(phi_hbm.at[dst_v.at[j]], b_v, sem_b)
        cp_a.wait()
        cp_b.wait()

        def mrow(i, carry):
            for c in range(D // 16):
                sl = pl.ds(c * 16, 16)
                p_v[i, sl] = a_v[i, sl] * b_v[i, sl]
            return carry

        lax.fori_loop(0, CHUNK, mrow, 0)
        pltpu.sync_copy(p_v, out_hbm.at[pl.ds(base + j * CHUNK, CHUNK)])


def _sc2_call(phi_sample, src3, dst3):
    mesh = plsc.VectorSubcoreMesh(core_axis_name="c", subcore_axis_name="s")
    return pl.kernel(
        _sc2_body,
        out_type=jax.ShapeDtypeStruct((E2, D), jnp.float32),
        mesh=mesh,
        scratch_types=[
            pltpu.VMEM((CPT, CHUNK), jnp.int32),
            pltpu.VMEM((CPT, CHUNK), jnp.int32),
            pltpu.VMEM((CHUNK, DP), jnp.float32),
            pltpu.VMEM((CHUNK, DP), jnp.float32),
            pltpu.VMEM((CHUNK, D), jnp.float32),
            pltpu.SemaphoreType.DMA,
            pltpu.SemaphoreType.DMA,
        ],
    )(phi_sample, src3, dst3)


def _tc3_body(prod_ref, pi_ref, bd_ref, out_ref):
    q = jnp.dot(prod_ref[...], pi_ref[...], preferred_element_type=jnp.float32,
                precision=_HIGHEST)
    m = jnp.max(q, axis=1, keepdims=True)
    e = jnp.exp(q - m)
    zz = (e / jnp.sum(e, axis=1, keepdims=True)).astype(jnp.bfloat16)
    out_ref[...] = jnp.dot(zz, bd_ref[...], preferred_element_type=jnp.float32)


def _tc3_call(prod, pi_wT, bd):
    return pl.pallas_call(
        _tc3_body,
        grid=(E2 // EB,),
        in_specs=[
            pl.BlockSpec((EB, D), lambda g: (g, 0)),
            pl.BlockSpec((D, C), lambda g: (0, 0)),
            pl.BlockSpec((C, N), lambda g: (0, 0)),
        ],
        out_specs=pl.BlockSpec((EB, N), lambda g: (g, 0)),
        out_shape=jax.ShapeDtypeStruct((E2, N), jnp.float32),
    )(prod, pi_wT, bd)


def kernel(edge_index, subject_idx, alpha_mean_w, s2p_w, s2p_b, s2b_w, s2b_b,
           wih_n, whh_n, bih_n, bhh_n, wih_c, whh_c, bih_c, bhh_c,
           bm_w, bm_b, bs_w, bs_b, pm_w, pm_b, ps_w, ps_b,
           pi_w, dec_w, eps_phi, eps_beta):
    alpha = lax.dynamic_index_in_dim(alpha_mean_w, subject_idx, axis=0,
                                     keepdims=True)
    src = jnp.concatenate([edge_index[0], edge_index[1]])
    dst = jnp.concatenate([edge_index[1], edge_index[0]])
    src3 = src.reshape(NW, CPT, CHUNK)
    dst3 = dst.reshape(NW, CPT, CHUNK)

    alpha_diag = jnp.kron(jnp.eye(D, dtype=jnp.float32), alpha.reshape(D, 1))
    phi_ext = _tc1_call(s2p_w.reshape(N, D * D), s2p_b.reshape(N, D),
                        alpha_diag)
    agg2 = _sc1_call(phi_ext, src3, dst3)
    phi_sample, bd = _tc2_call(
        phi_ext, agg2, alpha, s2b_w.reshape(C, D, D), s2b_b.reshape(C, D),
        wih_n.T, bih_n.reshape(1, 3 * D), bhh_n.reshape(1, 3 * D),
        wih_c.T, bih_c.reshape(1, 3 * D), bhh_c.reshape(1, 3 * D),
        pm_w.T, pm_b.reshape(1, D), ps_w.T, ps_b.reshape(1, D),
        bm_w.T, bm_b.reshape(1, D), bs_w.T, bs_b.reshape(1, D),
        eps_phi, eps_beta, dec_w.T)
    prod = _sc2_call(phi_sample, src3, dst3)
    return _tc3_call(prod, pi_w.T, bd)


# speedup vs baseline: 1.1105x; 1.1105x over previous
import jax
import jax.numpy as jnp
from jax import lax
from jax.experimental import pallas as pl
from jax.experimental.pallas import tpu as pltpu
from jax.experimental.pallas import tpu_sc as plsc

N = 2048
D = 64
C = 16
DP = 128
E2 = 32768
NW = 32
CPT = 8
CHUNK = 128
BN = 128
EB = 1024

_HIGHEST = lax.Precision.HIGHEST


def _sigmoid(x):
    return 1.0 / (1.0 + jnp.exp(-x))


def _softplus(x):
    return jnp.maximum(x, 0.0) + jnp.log(1.0 + jnp.exp(-jnp.abs(x)))


def _tc1_body(w_ref, b_ref, alpha_ref, out_ref):
    a = alpha_ref[...]
    val = jnp.sum(w_ref[...] * a[None], axis=2) + b_ref[...]
    ones = jnp.ones((BN, 1), jnp.float32)
    pad = jnp.zeros((BN, DP - D - 1), jnp.float32)
    out_ref[...] = jnp.concatenate([val, ones, pad], axis=1)


def _tc1_call(s2p_w3, s2p_b2, alpha):
    return pl.pallas_call(
        _tc1_body,
        grid=(N // BN,),
        in_specs=[
            pl.BlockSpec((BN, D, D), lambda g: (g, 0, 0)),
            pl.BlockSpec((BN, D), lambda g: (g, 0)),
            pl.BlockSpec((1, D), lambda g: (0, 0)),
        ],
        out_specs=pl.BlockSpec((BN, DP), lambda g: (g, 0)),
        out_shape=jax.ShapeDtypeStruct((N, DP), jnp.float32),
    )(s2p_w3, s2p_b2, alpha)


def _sc1_body(phi_hbm, src_hbm, dst_hbm, out_hbm, src_v, dst_v, buf, acc, sem):
    cid = lax.axis_index("c")
    sid = lax.axis_index("s")
    wid = cid * 16 + sid
    zero = jnp.zeros((16,), jnp.float32)

    def zrow(i, carry):
        for c in range(DP // 16):
            buf[i, pl.ds(c * 16, 16)] = zero
        return carry

    lax.fori_loop(0, CHUNK, zrow, 0)
    pltpu.sync_copy(buf, acc.at[pl.ds(sid * CHUNK, CHUNK)])
    pltpu.sync_copy(src_hbm.at[wid], src_v)
    pltpu.sync_copy(dst_hbm.at[wid], dst_v)
    plsc.subcore_barrier()
    for j in range(CPT):
        pltpu.async_copy(phi_hbm.at[src_v.at[j]], buf, sem).wait()
        pltpu.sync_copy(buf, acc.at[dst_v.at[j]], add=True)
    plsc.subcore_barrier()
    pltpu.sync_copy(acc.at[pl.ds(sid * CHUNK, CHUNK)], buf)
    pltpu.sync_copy(buf, out_hbm.at[cid, pl.ds(sid * CHUNK, CHUNK)])


def _sc1_call(phi_ext, src3, dst3):
    mesh = plsc.VectorSubcoreMesh(core_axis_name="c", subcore_axis_name="s")
    return pl.kernel(
        _sc1_body,
        out_type=jax.ShapeDtypeStruct((2, N, DP), jnp.float32),
        mesh=mesh,
        compiler_params=pltpu.CompilerParams(use_tc_tiling_on_sc=True),
        scratch_types=[
            pltpu.VMEM((CPT, CHUNK), jnp.int32),
            pltpu.VMEM((CPT, CHUNK), jnp.int32),
            pltpu.VMEM((CHUNK, DP), jnp.float32),
            pltpu.VMEM_SHARED((N, DP), jnp.float32),
            pltpu.SemaphoreType.DMA,
        ],
    )(phi_ext, src3, dst3)


def _tc2_body(phi_ext_ref, agg2_ref, alpha_ref, s2b_w_ref, s2b_b_ref,
              wihn_ref, bihn_ref, bhhn_ref, wihc_ref, bihc_ref, bhhc_ref,
              pm_ref, pmb_ref, ps_ref, psb_ref, bm_ref, bmb_ref, bs_ref,
              bsb_ref, epsp_ref, epsb_ref, dec_ref, phis_ref, bd_ref):
    phi_prior = phi_ext_ref[:, :D]
    acc = agg2_ref[0] + agg2_ref[1]
    agg = acc[:, :D]
    deg = acc[:, D:D + 1]
    ctx = jnp.where(deg == 0.0, phi_prior, agg) / jnp.maximum(deg, 1.0)

    x = jnp.concatenate([phi_prior, ctx], axis=1)
    gi = jnp.dot(x, wihn_ref[...], preferred_element_type=jnp.float32,
                 precision=_HIGHEST) + bihn_ref[...]
    bhh = bhhn_ref[...]
    r = _sigmoid(gi[:, :D] + bhh[:, :D])
    z = _sigmoid(gi[:, D:2 * D] + bhh[:, D:2 * D])
    n = jnp.tanh(gi[:, 2 * D:] + r * bhh[:, 2 * D:])
    h_phi = (1.0 - z) * n
    phi_mean = jnp.dot(h_phi, pm_ref[...], preferred_element_type=jnp.float32,
                       precision=_HIGHEST) + pmb_ref[...]
    phi_std = _softplus(jnp.dot(h_phi, ps_ref[...],
                                preferred_element_type=jnp.float32,
                                precision=_HIGHEST) + psb_ref[...])
    phis = phi_mean + phi_std * epsp_ref[...]
    phis_ref[...] = jnp.concatenate(
        [phis, jnp.zeros((N, DP - D), jnp.float32)], axis=1)

    bp = jnp.sum(s2b_w_ref[...] * alpha_ref[...][None], axis=2) + s2b_b_ref[...]
    xc = jnp.concatenate([bp, bp], axis=1)
    gic = jnp.dot(xc, wihc_ref[...], preferred_element_type=jnp.float32,
                  precision=_HIGHEST) + bihc_ref[...]
    bhc = bhhc_ref[...]
    rc = _sigmoid(gic[:, :D] + bhc[:, :D])
    zc = _sigmoid(gic[:, D:2 * D] + bhc[:, D:2 * D])
    nc = jnp.tanh(gic[:, 2 * D:] + rc * bhc[:, 2 * D:])
    h_beta = (1.0 - zc) * nc
    beta_mean = jnp.dot(h_beta, bm_ref[...], preferred_element_type=jnp.float32,
                        precision=_HIGHEST) + bmb_ref[...]
    beta_std = _softplus(jnp.dot(h_beta, bs_ref[...],
                                 preferred_element_type=jnp.float32,
                                 precision=_HIGHEST) + bsb_ref[...])
    beta_sample = beta_mean + beta_std * epsb_ref[...]
    bd_ref[...] = jnp.dot(beta_sample, dec_ref[...],
                          preferred_element_type=jnp.float32,
                          precision=_HIGHEST).astype(jnp.bfloat16)


def _tc2_call(phi_ext, agg2, alpha, s2b_w3, s2b_b2, wih_nT, bih_n2, bhh_n2,
              wih_cT, bih_c2, bhh_c2, pm_wT, pm_b2, ps_wT, ps_b2, bm_wT,
              bm_b2, bs_wT, bs_b2, eps_phi, eps_beta, dec_wT):
    return pl.pallas_call(
        _tc2_body,
        out_shape=(
            jax.ShapeDtypeStruct((N, DP), jnp.float32),
            jax.ShapeDtypeStruct((C, N), jnp.bfloat16),
        ),
    )(phi_ext, agg2, alpha, s2b_w3, s2b_b2, wih_nT, bih_n2, bhh_n2,
      wih_cT, bih_c2, bhh_c2, pm_wT, pm_b2, ps_wT, ps_b2, bm_wT, bm_b2,
      bs_wT, bs_b2, eps_phi, eps_beta, dec_wT)


def _sc2_body(phi_hbm, src_hbm, dst_hbm, out_hbm, src_v, dst_v, a_v, b_v,
              p_v, sem_a, sem_b):
    cid = lax.axis_index("c")
    sid = lax.axis_index("s")
    wid = cid * 16 + sid
    pltpu.sync_copy(src_hbm.at[wid], src_v)
    pltpu.sync_copy(dst_hbm.at[wid], dst_v)
    base = wid * (CPT * CHUNK)
    for j in range(CPT):
        cp_a = pltpu.async_copy(phi_hbm.at[src_v.at[j]], a_v, sem_a)
        cp_b = pltpu.async_copy(phi_hbm.at[dst_v.at[j]], b_v, sem_b)
        cp_a.wait()
        cp_b.wait()

        def mrow(i, carry):
            for c in range(D // 16):
                sl = pl.ds(c * 16, 16)
                p_v[i, sl] = a_v[i, sl] * b_v[i, sl]
            return carry

        lax.fori_loop(0, CHUNK, mrow, 0)
        pltpu.sync_copy(p_v, out_hbm.at[pl.ds(base + j * CHUNK, CHUNK)])


def _sc2_call(phi_sample, src3, dst3):
    mesh = plsc.VectorSubcoreMesh(core_axis_name="c", subcore_axis_name="s")
    return pl.kernel(
        _sc2_body,
        out_type=jax.ShapeDtypeStruct((E2, D), jnp.float32),
        mesh=mesh,
        compiler_params=pltpu.CompilerParams(use_tc_tiling_on_sc=True),
        scratch_types=[
            pltpu.VMEM((CPT, CHUNK), jnp.int32),
            pltpu.VMEM((CPT, CHUNK), jnp.int32),
            pltpu.VMEM((CHUNK, DP), jnp.float32),
            pltpu.VMEM((CHUNK, DP), jnp.float32),
            pltpu.VMEM((CHUNK, D), jnp.float32),
            pltpu.SemaphoreType.DMA,
            pltpu.SemaphoreType.DMA,
        ],
    )(phi_sample, src3, dst3)


def _tc3_body(prod_ref, pi_ref, bd_ref, out_ref):
    q = jnp.dot(prod_ref[...], pi_ref[...], preferred_element_type=jnp.float32,
                precision=_HIGHEST)
    m = jnp.max(q, axis=1, keepdims=True)
    e = jnp.exp(q - m)
    zz = (e / jnp.sum(e, axis=1, keepdims=True)).astype(jnp.bfloat16)
    out_ref[...] = jnp.dot(zz, bd_ref[...], preferred_element_type=jnp.float32)


def _tc3_call(prod, pi_wT, bd):
    return pl.pallas_call(
        _tc3_body,
        grid=(E2 // EB,),
        in_specs=[
            pl.BlockSpec((EB, D), lambda g: (g, 0)),
            pl.BlockSpec((D, C), lambda g: (0, 0)),
            pl.BlockSpec((C, N), lambda g: (0, 0)),
        ],
        out_specs=pl.BlockSpec((EB, N), lambda g: (g, 0)),
        out_shape=jax.ShapeDtypeStruct((E2, N), jnp.float32),
    )(prod, pi_wT, bd)


def kernel(edge_index, subject_idx, alpha_mean_w, s2p_w, s2p_b, s2b_w, s2b_b,
           wih_n, whh_n, bih_n, bhh_n, wih_c, whh_c, bih_c, bhh_c,
           bm_w, bm_b, bs_w, bs_b, pm_w, pm_b, ps_w, ps_b,
           pi_w, dec_w, eps_phi, eps_beta):
    alpha = lax.dynamic_index_in_dim(alpha_mean_w, subject_idx, axis=0,
                                     keepdims=True)
    src = jnp.concatenate([edge_index[0], edge_index[1]])
    dst = jnp.concatenate([edge_index[1], edge_index[0]])
    src3 = src.reshape(NW, CPT, CHUNK)
    dst3 = dst.reshape(NW, CPT, CHUNK)

    phi_ext = _tc1_call(s2p_w.reshape(N, D, D), s2p_b.reshape(N, D), alpha)
    agg2 = _sc1_call(phi_ext, src3, dst3)
    phi_sample, bd = _tc2_call(
        phi_ext, agg2, alpha, s2b_w.reshape(C, D, D), s2b_b.reshape(C, D),
        wih_n.T, bih_n.reshape(1, 3 * D), bhh_n.reshape(1, 3 * D),
        wih_c.T, bih_c.reshape(1, 3 * D), bhh_c.reshape(1, 3 * D),
        pm_w.T, pm_b.reshape(1, D), ps_w.T, ps_b.reshape(1, D),
        bm_w.T, bm_b.reshape(1, D), bs_w.T, bs_b.reshape(1, D),
        eps_phi, eps_beta, dec_w.T)
    prod = _sc2_call(phi_sample, src3, dst3)
    return _tc3_call(prod, pi_w.T, bd)


# speedup vs baseline: 1.1278x; 1.0156x over previous
import jax
import jax.numpy as jnp
from jax import lax
from jax.experimental import pallas as pl
from jax.experimental.pallas import tpu as pltpu
from jax.experimental.pallas import tpu_sc as plsc

N = 2048
D = 64
C = 16
DP = 128
E2 = 32768
NW = 32
CPT = 8
CPT2 = CPT // 2
E2H = E2 // 2
CHUNK = 128
BN = 128
EB = 1024

_HIGHEST = lax.Precision.HIGHEST


def _sigmoid(x):
    return 1.0 / (1.0 + jnp.exp(-x))


def _softplus(x):
    return jnp.maximum(x, 0.0) + jnp.log(1.0 + jnp.exp(-jnp.abs(x)))


def _tc1_body(w_ref, b_ref, alpha_ref, out_ref):
    a = alpha_ref[...]
    val = jnp.sum(w_ref[...] * a[None], axis=2) + b_ref[...]
    ones = jnp.ones((BN, 1), jnp.float32)
    pad = jnp.zeros((BN, DP - D - 1), jnp.float32)
    out_ref[...] = jnp.concatenate([val, ones, pad], axis=1)


def _tc1_call(s2p_w3, s2p_b2, alpha):
    return pl.pallas_call(
        _tc1_body,
        grid=(N // BN,),
        in_specs=[
            pl.BlockSpec((BN, D, D), lambda g: (g, 0, 0)),
            pl.BlockSpec((BN, D), lambda g: (g, 0)),
            pl.BlockSpec((1, D), lambda g: (0, 0)),
        ],
        out_specs=pl.BlockSpec((BN, DP), lambda g: (g, 0)),
        out_shape=jax.ShapeDtypeStruct((N, DP), jnp.float32),
    )(s2p_w3, s2p_b2, alpha)


def _prep_body(ei_ref, sa_ref, sb_ref):
    sa_ref[...] = ei_ref[0].reshape(NW, CPT2, CHUNK)
    sb_ref[...] = ei_ref[1].reshape(NW, CPT2, CHUNK)


def _prep_call(edge_index):
    return pl.pallas_call(
        _prep_body,
        out_shape=(
            jax.ShapeDtypeStruct((NW, CPT2, CHUNK), jnp.int32),
            jax.ShapeDtypeStruct((NW, CPT2, CHUNK), jnp.int32),
        ),
    )(edge_index)


def _sc1_body(phi_hbm, sa_hbm, sb_hbm, out_hbm, ia_v, ib_v, buf, acc, sem):
    cid = lax.axis_index("c")
    sid = lax.axis_index("s")
    wid = cid * 16 + sid
    zero = jnp.zeros((16,), jnp.float32)

    def zrow(i, carry):
        for c in range(DP // 16):
            buf[i, pl.ds(c * 16, 16)] = zero
        return carry

    lax.fori_loop(0, CHUNK, zrow, 0)
    pltpu.sync_copy(buf, acc.at[pl.ds(sid * CHUNK, CHUNK)])
    pltpu.sync_copy(sa_hbm.at[wid], ia_v)
    pltpu.sync_copy(sb_hbm.at[wid], ib_v)
    plsc.subcore_barrier()
    for j in range(CPT2):
        pltpu.async_copy(phi_hbm.at[ia_v.at[j]], buf, sem).wait()
        pltpu.sync_copy(buf, acc.at[ib_v.at[j]], add=True)
    for j in range(CPT2):
        pltpu.async_copy(phi_hbm.at[ib_v.at[j]], buf, sem).wait()
        pltpu.sync_copy(buf, acc.at[ia_v.at[j]], add=True)
    plsc.subcore_barrier()
    pltpu.sync_copy(acc.at[pl.ds(sid * CHUNK, CHUNK)], buf)
    pltpu.sync_copy(buf, out_hbm.at[cid, pl.ds(sid * CHUNK, CHUNK)])


def _sc1_call(phi_ext, sa, sb):
    mesh = plsc.VectorSubcoreMesh(core_axis_name="c", subcore_axis_name="s")
    return pl.kernel(
        _sc1_body,
        out_type=jax.ShapeDtypeStruct((2, N, DP), jnp.float32),
        mesh=mesh,
        compiler_params=pltpu.CompilerParams(use_tc_tiling_on_sc=True),
        scratch_types=[
            pltpu.VMEM((CPT2, CHUNK), jnp.int32),
            pltpu.VMEM((CPT2, CHUNK), jnp.int32),
            pltpu.VMEM((CHUNK, DP), jnp.float32),
            pltpu.VMEM_SHARED((N, DP), jnp.float32),
            pltpu.SemaphoreType.DMA,
        ],
    )(phi_ext, sa, sb)


def _tc2_body(phi_ext_ref, agg2_ref, alpha_ref, s2b_w_ref, s2b_b_ref,
              wihn_ref, bihn_ref, bhhn_ref, wihc_ref, bihc_ref, bhhc_ref,
              pm_ref, pmb_ref, ps_ref, psb_ref, bm_ref, bmb_ref, bs_ref,
              bsb_ref, epsp_ref, epsb_ref, dec_ref, phis_ref, bd_ref):
    phi_prior = phi_ext_ref[:, :D]
    acc = agg2_ref[0] + agg2_ref[1]
    agg = acc[:, :D]
    deg = acc[:, D:D + 1]
    ctx = jnp.where(deg == 0.0, phi_prior, agg) / jnp.maximum(deg, 1.0)

    x = jnp.concatenate([phi_prior, ctx], axis=1)
    gi = jnp.dot(x, wihn_ref[...], preferred_element_type=jnp.float32,
                 precision=_HIGHEST) + bihn_ref[...]
    bhh = bhhn_ref[...]
    r = _sigmoid(gi[:, :D] + bhh[:, :D])
    z = _sigmoid(gi[:, D:2 * D] + bhh[:, D:2 * D])
    n = jnp.tanh(gi[:, 2 * D:] + r * bhh[:, 2 * D:])
    h_phi = (1.0 - z) * n
    phi_mean = jnp.dot(h_phi, pm_ref[...], preferred_element_type=jnp.float32,
                       precision=_HIGHEST) + pmb_ref[...]
    phi_std = _softplus(jnp.dot(h_phi, ps_ref[...],
                                preferred_element_type=jnp.float32,
                                precision=_HIGHEST) + psb_ref[...])
    phis = phi_mean + phi_std * epsp_ref[...]
    phis_ref[...] = jnp.concatenate(
        [phis, jnp.zeros((N, DP - D), jnp.float32)], axis=1)

    bp = jnp.sum(s2b_w_ref[...] * alpha_ref[...][None], axis=2) + s2b_b_ref[...]
    xc = jnp.concatenate([bp, bp], axis=1)
    gic = jnp.dot(xc, wihc_ref[...], preferred_element_type=jnp.float32,
                  precision=_HIGHEST) + bihc_ref[...]
    bhc = bhhc_ref[...]
    rc = _sigmoid(gic[:, :D] + bhc[:, :D])
    zc = _sigmoid(gic[:, D:2 * D] + bhc[:, D:2 * D])
    nc = jnp.tanh(gic[:, 2 * D:] + rc * bhc[:, 2 * D:])
    h_beta = (1.0 - zc) * nc
    beta_mean = jnp.dot(h_beta, bm_ref[...], preferred_element_type=jnp.float32,
                        precision=_HIGHEST) + bmb_ref[...]
    beta_std = _softplus(jnp.dot(h_beta, bs_ref[...],
                                 preferred_element_type=jnp.float32,
                                 precision=_HIGHEST) + bsb_ref[...])
    beta_sample = beta_mean + beta_std * epsb_ref[...]
    bd_ref[...] = jnp.dot(beta_sample, dec_ref[...],
                          preferred_element_type=jnp.float32,
                          precision=_HIGHEST).astype(jnp.bfloat16)


def _tc2_call(phi_ext, agg2, alpha, s2b_w3, s2b_b2, wih_nT, bih_n2, bhh_n2,
              wih_cT, bih_c2, bhh_c2, pm_wT, pm_b2, ps_wT, ps_b2, bm_wT,
              bm_b2, bs_wT, bs_b2, eps_phi, eps_beta, dec_wT):
    return pl.pallas_call(
        _tc2_body,
        out_shape=(
            jax.ShapeDtypeStruct((N, DP), jnp.float32),
            jax.ShapeDtypeStruct((C, N), jnp.bfloat16),
        ),
    )(phi_ext, agg2, alpha, s2b_w3, s2b_b2, wih_nT, bih_n2, bhh_n2,
      wih_cT, bih_c2, bhh_c2, pm_wT, pm_b2, ps_wT, ps_b2, bm_wT, bm_b2,
      bs_wT, bs_b2, eps_phi, eps_beta, dec_wT)


def _sc2_body(phi_hbm, src_hbm, dst_hbm, out_hbm, src_v, dst_v, a_v, b_v,
              p_v, sem_a, sem_b):
    cid = lax.axis_index("c")
    sid = lax.axis_index("s")
    wid = cid * 16 + sid
    pltpu.sync_copy(src_hbm.at[wid], src_v)
    pltpu.sync_copy(dst_hbm.at[wid], dst_v)
    base = wid * (CPT2 * CHUNK)
    for j in range(CPT2):
        cp_a = pltpu.async_copy(phi_hbm.at[src_v.at[j]], a_v, sem_a)
        cp_b = pltpu.async_copy(phi_hbm.at[dst_v.at[j]], b_v, sem_b)
        cp_a.wait()
        cp_b.wait()

        def mrow(i, carry):
            for c in range(D // 16):
                sl = pl.ds(c * 16, 16)
                p_v[i, sl] = a_v[i, sl] * b_v[i, sl]
            return carry

        lax.fori_loop(0, CHUNK, mrow, 0)
        pltpu.sync_copy(p_v, out_hbm.at[pl.ds(base + j * CHUNK, CHUNK)])


def _sc2_call(phi_sample, src_half, dst_half):
    mesh = plsc.VectorSubcoreMesh(core_axis_name="c", subcore_axis_name="s")
    return pl.kernel(
        _sc2_body,
        out_type=jax.ShapeDtypeStruct((E2H, D), jnp.float32),
        mesh=mesh,
        compiler_params=pltpu.CompilerParams(use_tc_tiling_on_sc=True),
        scratch_types=[
            pltpu.VMEM((CPT2, CHUNK), jnp.int32),
            pltpu.VMEM((CPT2, CHUNK), jnp.int32),
            pltpu.VMEM((CHUNK, DP), jnp.float32),
            pltpu.VMEM((CHUNK, DP), jnp.float32),
            pltpu.VMEM((CHUNK, D), jnp.float32),
            pltpu.SemaphoreType.DMA,
            pltpu.SemaphoreType.DMA,
        ],
    )(phi_sample, src_half, dst_half)


def _tc3_body(prod_ref, pi_ref, bd_ref, out_ref):
    q = jnp.dot(prod_ref[...], pi_ref[...], preferred_element_type=jnp.float32,
                precision=_HIGHEST)
    m = jnp.max(q, axis=1, keepdims=True)
    e = jnp.exp(q - m)
    zz = (e / jnp.sum(e, axis=1, keepdims=True)).astype(jnp.bfloat16)
    out_ref[...] = jnp.dot(zz, bd_ref[...], preferred_element_type=jnp.float32)


def _tc3_body_b(prod_ref, pi_ref, bd_ref, prev_ref, out_ref):
    del prev_ref
    _tc3_body(prod_ref, pi_ref, bd_ref, out_ref)


def _tc3_call_a(prod_a, pi_wT, bd):
    return pl.pallas_call(
        _tc3_body,
        grid=(E2H // EB,),
        in_specs=[
            pl.BlockSpec((EB, D), lambda g: (g, 0)),
            pl.BlockSpec((D, C), lambda g: (0, 0)),
            pl.BlockSpec((C, N), lambda g: (0, 0)),
        ],
        out_specs=pl.BlockSpec((EB, N), lambda g: (g, 0)),
        out_shape=jax.ShapeDtypeStruct((E2, N), jnp.float32),
    )(prod_a, pi_wT, bd)


def _tc3_call_b(prod_b, pi_wT, bd, out_prev):
    nblk = E2H // EB
    return pl.pallas_call(
        _tc3_body_b,
        grid=(nblk,),
        in_specs=[
            pl.BlockSpec((EB, D), lambda g: (g, 0)),
            pl.BlockSpec((D, C), lambda g: (0, 0)),
            pl.BlockSpec((C, N), lambda g: (0, 0)),
            pl.BlockSpec(memory_space=pl.ANY),
        ],
        out_specs=pl.BlockSpec((EB, N), lambda g: (g + nblk, 0)),
        out_shape=jax.ShapeDtypeStruct((E2, N), jnp.float32),
        input_output_aliases={3: 0},
    )(prod_b, pi_wT, bd, out_prev)


def kernel(edge_index, subject_idx, alpha_mean_w, s2p_w, s2p_b, s2b_w, s2b_b,
           wih_n, whh_n, bih_n, bhh_n, wih_c, whh_c, bih_c, bhh_c,
           bm_w, bm_b, bs_w, bs_b, pm_w, pm_b, ps_w, ps_b,
           pi_w, dec_w, eps_phi, eps_beta):
    alpha = lax.dynamic_index_in_dim(alpha_mean_w, subject_idx, axis=0,
                                     keepdims=True)
    sa, sb = _prep_call(edge_index)

    phi_ext = _tc1_call(s2p_w.reshape(N, D, D), s2p_b.reshape(N, D), alpha)
    agg2 = _sc1_call(phi_ext, sa, sb)
    phi_sample, bd = _tc2_call(
        phi_ext, agg2, alpha, s2b_w.reshape(C, D, D), s2b_b.reshape(C, D),
        wih_n.T, bih_n.reshape(1, 3 * D), bhh_n.reshape(1, 3 * D),
        wih_c.T, bih_c.reshape(1, 3 * D), bhh_c.reshape(1, 3 * D),
        pm_w.T, pm_b.reshape(1, D), ps_w.T, ps_b.reshape(1, D),
        bm_w.T, bm_b.reshape(1, D), bs_w.T, bs_b.reshape(1, D),
        eps_phi, eps_beta, dec_w.T)
    prod_a = _sc2_call(phi_sample, sa, sb)
    prod_b = _sc2_call(phi_sample, sb, sa)
    out_a = _tc3_call_a(prod_a, pi_w.T, bd)
    return _tc3_call_b(prod_b, pi_w.T, bd, out_a)
